# baseline (speedup 1.0000x reference)
import jax
import jax.numpy as jnp
from jax.experimental import pallas as pl
from jax.experimental.pallas import tpu as pltpu

_N_CODES = 1024
_D = 256
_BT = 4096
_N_TOK = 32768
_GRID = _N_TOK // _BT
_BETA = 0.25


def _phase1(zr_ref, e_ref, e2_ref, idx_ref, scr_ref):
    zr = zr_ref[...]
    e = e_ref[...]
    s2 = jax.lax.dot_general(
        zr, e, (((1,), (1,)), ((), ())),
        preferred_element_type=jnp.float32)
    z2 = jnp.sum(zr * zr, axis=1, keepdims=True)
    e2 = e2_ref[0:1, :]
    d = (z2 + e2) - s2
    m = jnp.min(d, axis=1, keepdims=True)
    iotaf = jax.lax.broadcasted_iota(
        jnp.int32, (_BT, _N_CODES), 1).astype(jnp.float32)
    candf = jnp.where(d == m, iotaf, float(_N_CODES))
    idx = jnp.min(candf, axis=1).astype(jnp.int32)
    idx_ref[0, 0, :] = idx
    scr_ref[0] = idx.reshape(128, 32)


def _phase2(zr_ref, et_ref, scr_ref, out_ref, loss_ref, acc_ref):
    t = pl.program_id(0)
    i_loc = t % 2
    et = et_ref[...]
    sf = scr_ref[0].astype(jnp.float32)
    jlane = jax.lax.broadcasted_iota(jnp.int32, (1, 32), 1)
    iok = jax.lax.broadcasted_iota(
        jnp.int32, (_D, _N_CODES), 1).astype(jnp.float32)
    halves = []
    for q in range(16):
        sel = (jlane == 16 * i_loc + q).astype(jnp.float32)
        col = jnp.sum(sf * sel, axis=1, keepdims=True)
        oht = (iok == col).astype(jnp.float32)
        g = jax.lax.dot_general(
            et, oht, (((1,), (1,)), ((), ())),
            preferred_element_type=jnp.float32)
        halves.append(g)
    out = jnp.concatenate(halves, axis=0)
    out_ref[...] = out

    @pl.when(t == 0)
    def _init():
        acc_ref[0] = 0.0

    diff = out - zr_ref[...]
    acc_ref[0] += jnp.sum(diff * diff)

    @pl.when(t == _GRID - 1)
    def _fin():
        val = acc_ref[0] * ((1.0 + _BETA) / (_N_TOK * _D))
        loss_ref[...] = jnp.full((1, 1), val, dtype=jnp.float32)


def kernel(z, embedding):
    zrow = jnp.transpose(z, (0, 2, 3, 4, 1)).reshape(_N_TOK, _D)
    emb2 = embedding + embedding
    et = embedding.T
    e2 = jnp.sum(embedding ** 2, axis=1)
    e2b = jnp.broadcast_to(e2[None, :], (8, _N_CODES))

    idx3, scr = pl.pallas_call(
        _phase1,
        grid=(_GRID,),
        in_specs=[
            pl.BlockSpec((_BT, _D), lambda t: (t, 0)),
            pl.BlockSpec((_N_CODES, _D), lambda t: (0, 0)),
            pl.BlockSpec((8, _N_CODES), lambda t: (0, 0)),
        ],
        out_specs=[
            pl.BlockSpec((1, 1, _BT), lambda t: (t, 0, 0)),
            pl.BlockSpec((1, 128, 32), lambda t: (t // 2, t % 2, 0)),
        ],
        out_shape=[
            jax.ShapeDtypeStruct((_GRID, 1, _BT), jnp.int32),
            jax.ShapeDtypeStruct((4, 256, 32), jnp.int32),
        ],
    )(zrow, emb2, e2b)

    out5f, loss = pl.pallas_call(
        _phase2,
        grid=(_GRID,),
        in_specs=[
            pl.BlockSpec((_BT, _D), lambda t: (t, 0)),
            pl.BlockSpec((_D, _N_CODES), lambda t: (0, 0)),
            pl.BlockSpec((1, 256, 32), lambda t: (t // 2, 0, 0)),
        ],
        out_specs=[
            pl.BlockSpec((_BT, _D), lambda t: (t, 0)),
            pl.BlockSpec((1, 1), lambda t: (0, 0)),
        ],
        out_shape=[
            jax.ShapeDtypeStruct((_N_TOK, _D), jnp.float32),
            jax.ShapeDtypeStruct((1, 1), jnp.float32),
        ],
        scratch_shapes=[pltpu.SMEM((1,), jnp.float32)],
    )(zrow, et, scr)

    z_q_out = jnp.transpose(out5f.reshape(4, 8, 32, 32, _D), (0, 4, 1, 2, 3))
    encoding_indices = idx3.reshape(_N_TOK)
    vq_loss = loss.reshape(())
    return (z_q_out, vq_loss, encoding_indices)


# speedup vs baseline: 1.0389x; 1.0389x over previous
import jax
import jax.numpy as jnp
from jax.experimental import pallas as pl
from jax.experimental.pallas import tpu as pltpu

_N_CODES = 1024
_D = 256
_BT = 4096
_N_TOK = 32768
_GRID = _N_TOK // _BT
_BETA = 0.25


def _vq(zr_ref, e_ref, et_ref, e2_ref, idx_ref, out_ref, loss_ref,
        scr_ref, acc_ref):
    t = pl.program_id(0)

    @pl.when(t < _GRID)
    def _phase1():
        zr = zr_ref[...]
        e = e_ref[...]
        s = jax.lax.dot_general(
            zr, e, (((1,), (1,)), ((), ())),
            preferred_element_type=jnp.float32)
        z2 = jnp.sum(zr * zr, axis=1, keepdims=True)
        e2 = e2_ref[0:1, :]
        d = (z2 + e2) - 2.0 * s
        m = jnp.min(d, axis=1, keepdims=True)
        iotaf = jax.lax.broadcasted_iota(
            jnp.int32, (_BT, _N_CODES), 1).astype(jnp.float32)
        candf = jnp.where(d == m, iotaf, float(_N_CODES))
        idx = jnp.min(candf, axis=1).astype(jnp.int32)
        idx_ref[0, 0, :] = idx
        b = t // 2
        i_loc = t % 2
        scr_ref[pl.ds(b, 1), pl.ds(128 * i_loc, 128), :] = (
            idx.reshape(1, 128, 32))

    @pl.when(t >= _GRID)
    def _phase2():
        tp = t - _GRID
        b = tp // 2
        i_loc = tp % 2
        et = et_ref[...]
        sf = scr_ref[pl.ds(b, 1), :, :][0].astype(jnp.float32)
        jlane = jax.lax.broadcasted_iota(jnp.int32, (1, 32), 1)
        iok = jax.lax.broadcasted_iota(
            jnp.int32, (_D, _N_CODES), 1).astype(jnp.float32)
        halves = []
        for q in range(16):
            sel = (jlane == 16 * i_loc + q).astype(jnp.float32)
            col = jnp.sum(sf * sel, axis=1, keepdims=True)
            oht = (iok == col).astype(jnp.float32)
            g = jax.lax.dot_general(
                et, oht, (((1,), (1,)), ((), ())),
                preferred_element_type=jnp.float32)
            halves.append(g)
        out = jnp.concatenate(halves, axis=0)
        out_ref[...] = out

        @pl.when(tp == 0)
        def _init():
            acc_ref[0] = 0.0

        diff = out - zr_ref[...]
        acc_ref[0] += jnp.sum(diff * diff)

        @pl.when(tp == _GRID - 1)
        def _fin():
            val = acc_ref[0] * ((1.0 + _BETA) / (_N_TOK * _D))
            loss_ref[...] = jnp.full((1, 1), val, dtype=jnp.float32)


def kernel(z, embedding):
    zrow = jnp.transpose(z, (0, 2, 3, 4, 1)).reshape(_N_TOK, _D)
    et = embedding.T
    e2 = jnp.sum(embedding ** 2, axis=1)
    e2b = jnp.broadcast_to(e2[None, :], (8, _N_CODES))

    idx3, out5f, loss = pl.pallas_call(
        _vq,
        grid=(2 * _GRID,),
        in_specs=[
            pl.BlockSpec((_BT, _D), lambda t: (t % _GRID, 0)),
            pl.BlockSpec((_N_CODES, _D), lambda t: (0, 0)),
            pl.BlockSpec((_D, _N_CODES), lambda t: (0, 0)),
            pl.BlockSpec((8, _N_CODES), lambda t: (0, 0)),
        ],
        out_specs=[
            pl.BlockSpec((1, 1, _BT),
                         lambda t: (jnp.minimum(t, _GRID - 1), 0, 0)),
            pl.BlockSpec((_BT, _D),
                         lambda t: (jnp.maximum(t - _GRID, 0), 0)),
            pl.BlockSpec((1, 1), lambda t: (0, 0)),
        ],
        out_shape=[
            jax.ShapeDtypeStruct((_GRID, 1, _BT), jnp.int32),
            jax.ShapeDtypeStruct((_N_TOK, _D), jnp.float32),
            jax.ShapeDtypeStruct((1, 1), jnp.float32),
        ],
        scratch_shapes=[
            pltpu.VMEM((4, 256, 32), jnp.int32),
            pltpu.SMEM((1,), jnp.float32),
        ],
    )(zrow, embedding, et, e2b)

    z_q_out = jnp.transpose(out5f.reshape(4, 8, 32, 32, _D), (0, 4, 1, 2, 3))
    encoding_indices = idx3.reshape(_N_TOK)
    vq_loss = loss.reshape(())
    return (z_q_out, vq_loss, encoding_indices)
